# baseline (speedup 1.0000x reference)
import jax
import jax.numpy as jnp
from jax.experimental import pallas as pl
from jax.experimental.pallas import tpu as pltpu

N = 4096
D = 128
C = 40
BR = 512
NBLK = N // BR


def _kernel(
    adj_ref, x_ref, we1_ref, we2_ref, wh1_ref, wh2_ref, wd_ref, wmlp_ref,
    sc_ref,
    e2_ref, hh2_ref, ln_ref, out_ref,
    abf_ref, d_ref, va_ref, vb_ref, acc_ref,
):
    p = pl.program_id(0)
    i = pl.program_id(1)
    la = sc_ref[0, 0]
    rows = pl.ds(i * BR, BR)

    @pl.when(p == 0)
    def _phase0():
        a = adj_ref[0]
        abf_ref[rows, :] = a.astype(jnp.bfloat16)
        deg = jnp.sum(a, axis=1, keepdims=True) + la
        d = jnp.where(deg > 0, jax.lax.rsqrt(deg), 0.0)
        d_ref[rows, :] = d.astype(jnp.bfloat16)
        xb = x_ref[0]
        xw_e = jnp.dot(xb, we1_ref[...], preferred_element_type=jnp.float32)
        xw_h = jnp.dot(xb, wh1_ref[...], preferred_element_type=jnp.float32)
        va_ref[rows, : D] = (d * xw_e).astype(jnp.bfloat16)
        va_ref[rows, D:] = (d * xw_h).astype(jnp.bfloat16)

        for s in range(NBLK):

            @pl.when(i == s)
            def _step(s=s):
                lo, hi = s * BR, (s + 1) * BR
                acc_ref[lo:hi, :] = jnp.dot(
                    abf_ref[lo:hi, :hi],
                    va_ref[:hi, :],
                    preferred_element_type=jnp.float32,
                )
                if s > 0:
                    acc_ref[:lo, :] += jnp.dot(
                        abf_ref[:lo, lo:hi],
                        va_ref[lo:hi, :],
                        preferred_element_type=jnp.float32,
                    )
                if s == NBLK - 1:
                    for k in range(NBLK):
                        klo, khi = k * BR, (k + 1) * BR
                        dk = d_ref[klo:khi, :].astype(jnp.float32)
                        ak = acc_ref[klo:khi, :] + la * va_ref[
                            klo:khi, :
                        ].astype(jnp.float32)
                        hk = jnp.maximum(dk * ak, 0.0).astype(jnp.bfloat16)
                        hw_e = jnp.dot(
                            hk[:, : D],
                            we2_ref[...].astype(jnp.bfloat16),
                            preferred_element_type=jnp.float32,
                        )
                        hw_h = jnp.dot(
                            hk[:, D:],
                            wh2_ref[...].astype(jnp.bfloat16),
                            preferred_element_type=jnp.float32,
                        )
                        vb_ref[klo:khi, : D] = (dk * hw_e).astype(jnp.bfloat16)
                        vb_ref[klo:khi, D:] = (dk * hw_h).astype(jnp.bfloat16)

    def _spmm(v):
        acc = jnp.dot(
            abf_ref[rows, :], v[...], preferred_element_type=jnp.float32
        )
        return acc + la * v[rows, :].astype(jnp.float32)

    @pl.when(p == 1)
    def _phase2():
        d = d_ref[rows, :].astype(jnp.float32)
        y = d * _spmm(vb_ref)
        e2 = y[:, : D]
        hh2 = y[:, D:]
        e2_ref[...] = e2
        hh2_ref[...] = hh2
        ln_ref[0] = (
            jnp.dot(
                e2.astype(jnp.bfloat16),
                wmlp_ref[...].astype(jnp.bfloat16),
                preferred_element_type=jnp.float32,
            )
            + sc_ref[:, 1:]
        )
        va_ref[rows, : C] = (
            d
            * jnp.dot(
                hh2.astype(jnp.bfloat16),
                wd_ref[...].astype(jnp.bfloat16),
                preferred_element_type=jnp.float32,
            )
        ).astype(jnp.bfloat16)

    @pl.when(p == 2)
    def _phase3():
        acc = jnp.dot(
            abf_ref[rows, :],
            va_ref[:, : C],
            preferred_element_type=jnp.float32,
        )
        acc = acc + la * va_ref[rows, : C].astype(jnp.float32)
        out_ref[0] = d_ref[rows, :].astype(jnp.float32) * acc


def _phase0_rows3(p, i):
    return (0, jnp.where(p == 0, i, NBLK - 1), 0)


def _phase2_rows3(p, i):
    return (0, jnp.where(p == 1, i, jnp.where(p < 1, 0, NBLK - 1)), 0)


def _phase2_rows(p, i):
    return (jnp.where(p == 1, i, jnp.where(p < 1, 0, NBLK - 1)), 0)


def _phase3_rows3(p, i):
    return (0, jnp.where(p == 2, i, 0), 0)


def _const(p, i):
    return (0, 0)


@jax.jit
def kernel(x, adj, loop_att, We1, We2, Wh1, Wh2, Wd, Wmlp, bmlp):
    sc = jnp.concatenate([jnp.reshape(loop_att, (1,)), bmlp])[None, :]

    e2, hh2, ln, outp = pl.pallas_call(
        _kernel,
        grid=(3, NBLK),
        in_specs=[
            pl.BlockSpec((1, BR, N), _phase0_rows3),
            pl.BlockSpec((1, BR, D), _phase0_rows3),
            pl.BlockSpec((D, D), _const),
            pl.BlockSpec((D, D), _const),
            pl.BlockSpec((D, D), _const),
            pl.BlockSpec((D, D), _const),
            pl.BlockSpec((D, C), _const),
            pl.BlockSpec((D, C), _const),
            pl.BlockSpec((1, 1 + C), _const),
        ],
        out_specs=[
            pl.BlockSpec((BR, D), _phase2_rows),
            pl.BlockSpec((BR, D), _phase2_rows),
            pl.BlockSpec((1, BR, C), _phase2_rows3),
            pl.BlockSpec((1, BR, C), _phase3_rows3),
        ],
        out_shape=[
            jax.ShapeDtypeStruct((N, D), jnp.float32),
            jax.ShapeDtypeStruct((N, D), jnp.float32),
            jax.ShapeDtypeStruct((1, N, C), jnp.float32),
            jax.ShapeDtypeStruct((1, N, C), jnp.float32),
        ],
        scratch_shapes=[
            pltpu.VMEM((N, N), jnp.bfloat16),
            pltpu.VMEM((N, 1), jnp.bfloat16),
            pltpu.VMEM((N, 2 * D), jnp.bfloat16),
            pltpu.VMEM((N, 2 * D), jnp.bfloat16),
            pltpu.VMEM((N, 2 * D), jnp.float32),
        ],
        compiler_params=pltpu.CompilerParams(
            vmem_limit_bytes=100 * 1024 * 1024,
        ),
    )(adj, x, We1, We2, Wh1, Wh2, Wd, Wmlp, sc)

    return (outp, ln, e2, hh2)


# speedup vs baseline: 1.0437x; 1.0437x over previous
import functools

import jax
import jax.numpy as jnp
from jax.experimental import pallas as pl
from jax.experimental.pallas import tpu as pltpu
from jax.experimental.layout import Format, Layout

N = 4096
D = 128
C = 40
BR = 512
NBLK = N // BR


def _kernel(
    adj_ref, x_ref, we1_ref, we2_ref, wh1_ref, wh2_ref, wdm_ref,
    bmlp_ref, la_ref,
    e2_ref, hh2_ref, ln_ref, out_ref,
    abf_ref, d_ref, va_ref, vb_ref, acc_ref,
):
    p = pl.program_id(0)
    i = pl.program_id(1)
    la = la_ref[0, 0]
    rows = pl.ds(i * BR, BR)

    @pl.when(p == 0)
    def _phase0():
        a = adj_ref[0]
        abf_ref[rows, :] = a.astype(jnp.bfloat16)
        deg = jnp.sum(a, axis=1, keepdims=True) + la
        d = jnp.where(deg > 0, jax.lax.rsqrt(deg), 0.0)
        d_ref[rows, :] = d.astype(jnp.bfloat16)
        xb = x_ref[0]
        xw_e = jnp.dot(xb, we1_ref[...], preferred_element_type=jnp.float32)
        xw_h = jnp.dot(xb, wh1_ref[...], preferred_element_type=jnp.float32)
        va_ref[rows, : D] = (d * xw_e).astype(jnp.bfloat16)
        va_ref[rows, D:] = (d * xw_h).astype(jnp.bfloat16)

        for s in range(NBLK):

            @pl.when(i == s)
            def _step(s=s):
                lo, hi = s * BR, (s + 1) * BR
                acc_ref[lo:hi, :] = jnp.dot(
                    abf_ref[lo:hi, :hi],
                    va_ref[:hi, :],
                    preferred_element_type=jnp.float32,
                )
                if s > 0:
                    acc_ref[:lo, :] += jnp.dot(
                        abf_ref[:lo, lo:hi],
                        va_ref[lo:hi, :],
                        preferred_element_type=jnp.float32,
                    )
                if s == NBLK - 1:
                    for k in range(NBLK):
                        klo, khi = k * BR, (k + 1) * BR
                        dk = d_ref[klo:khi, :].astype(jnp.float32)
                        ak = acc_ref[klo:khi, :] + la * va_ref[
                            klo:khi, :
                        ].astype(jnp.float32)
                        hk = jnp.maximum(dk * ak, 0.0).astype(jnp.bfloat16)
                        hw_e = jnp.dot(
                            hk[:, : D],
                            we2_ref[...].astype(jnp.bfloat16),
                            preferred_element_type=jnp.float32,
                        )
                        hw_h = jnp.dot(
                            hk[:, D:],
                            wh2_ref[...].astype(jnp.bfloat16),
                            preferred_element_type=jnp.float32,
                        )
                        vb_ref[klo:khi, : D] = (dk * hw_e).astype(jnp.bfloat16)
                        vb_ref[klo:khi, D:] = (dk * hw_h).astype(jnp.bfloat16)

    def _spmm(v):
        acc = jnp.dot(
            abf_ref[rows, :], v[...], preferred_element_type=jnp.float32
        )
        return acc + la * v[rows, :].astype(jnp.float32)

    @pl.when(p == 1)
    def _phase2():
        d = d_ref[rows, :].astype(jnp.float32)
        y = d * _spmm(vb_ref)
        e2 = y[:, : D]
        hh2 = y[:, D:]
        e2_ref[...] = e2
        hh2_ref[...] = hh2
        ln_ref[0] = (
            jnp.dot(
                e2.astype(jnp.bfloat16),
                wdm_ref[:, C:].astype(jnp.bfloat16),
                preferred_element_type=jnp.float32,
            )
            + bmlp_ref[...]
        )
        va_ref[rows, : C] = (
            d
            * jnp.dot(
                hh2.astype(jnp.bfloat16),
                wdm_ref[:, : C].astype(jnp.bfloat16),
                preferred_element_type=jnp.float32,
            )
        ).astype(jnp.bfloat16)

    @pl.when(p == 2)
    def _phase3():
        acc = jnp.dot(
            abf_ref[rows, :],
            va_ref[:, : C],
            preferred_element_type=jnp.float32,
        )
        acc = acc + la * va_ref[rows, : C].astype(jnp.float32)
        out_ref[0] = d_ref[rows, :].astype(jnp.float32) * acc


def _phase0_rows3(p, i):
    return (0, jnp.where(p == 0, i, NBLK - 1), 0)


def _phase2_rows3(p, i):
    return (0, jnp.where(p == 1, i, jnp.where(p < 1, 0, NBLK - 1)), 0)


def _phase2_rows(p, i):
    return (jnp.where(p == 1, i, jnp.where(p < 1, 0, NBLK - 1)), 0)


def _phase3_rows3(p, i):
    return (0, jnp.where(p == 2, i, 0), 0)


def _const(p, i):
    return (0, 0)


def _kernel_impl(x, adj, loop_att, We1, We2, Wh1, Wh2, Wd, Wmlp, bmlp):
    la = jnp.reshape(loop_att, (1, 1))

    e2, hh2, ln, outp = pl.pallas_call(
        _kernel,
        grid=(3, NBLK),
        in_specs=[
            pl.BlockSpec((1, BR, N), _phase0_rows3),
            pl.BlockSpec((1, BR, D), _phase0_rows3),
            pl.BlockSpec((D, D), _const),
            pl.BlockSpec((D, D), _const),
            pl.BlockSpec((D, D), _const),
            pl.BlockSpec((D, D), _const),
            pl.BlockSpec((D, 2 * C), _const),
            pl.BlockSpec((1, C), _const),
            pl.BlockSpec((1, 1), _const),
        ],
        out_specs=[
            pl.BlockSpec((BR, D), _phase2_rows),
            pl.BlockSpec((BR, D), _phase2_rows),
            pl.BlockSpec((1, BR, C), _phase2_rows3),
            pl.BlockSpec((1, BR, C), _phase3_rows3),
        ],
        out_shape=[
            jax.ShapeDtypeStruct((N, D), jnp.float32),
            jax.ShapeDtypeStruct((N, D), jnp.float32),
            jax.ShapeDtypeStruct((1, N, C), jnp.float32),
            jax.ShapeDtypeStruct((1, N, C), jnp.float32),
        ],
        scratch_shapes=[
            pltpu.VMEM((N, N), jnp.bfloat16),
            pltpu.VMEM((N, 1), jnp.bfloat16),
            pltpu.VMEM((N, 2 * D), jnp.bfloat16),
            pltpu.VMEM((N, 2 * D), jnp.bfloat16),
            pltpu.VMEM((N, 2 * D), jnp.float32),
        ],
        compiler_params=pltpu.CompilerParams(
            vmem_limit_bytes=100 * 1024 * 1024,
        ),
    )(adj, x, We1, We2, Wh1, Wh2, jnp.concatenate([Wd, Wmlp], axis=1),
      bmlp[None, :], la)

    return (outp, ln, e2, hh2)


_jitted = None


def kernel(x, adj, loop_att, We1, We2, Wh1, Wh2, Wd, Wmlp, bmlp):
    global _jitted
    if _jitted is None:
        sh = jax.sharding.SingleDeviceSharding(jax.devices()[0])
        f3 = Format(Layout(major_to_minor=(0, 1, 2)), sh)
        f2 = Format(Layout(major_to_minor=(0, 1)), sh)
        _jitted = jax.jit(_kernel_impl, out_shardings=(f3, f3, f2, f2))
    return _jitted(x, adj, loop_att, We1, We2, Wh1, Wh2, Wd, Wmlp, bmlp)
